# baseline (speedup 1.0000x reference)
import functools
import math

import jax
import jax.numpy as jnp
from jax import lax
from jax.experimental import pallas as pl
from jax.experimental.pallas import tpu as pltpu
from jax.experimental.pallas import tpu_sc as plsc

D_MODEL_K = 64
VOCAB_K = 1_000_000
SCALE = math.sqrt(D_MODEL_K)

NC = 2
NS = 16
NW = NC * NS

B_TOTAL = 4096 * 200
B_PER_W = B_TOTAL // NW
CHUNK = 512
NITER = B_PER_W // CHUNK


@functools.partial(
    pl.kernel,
    out_type=jax.ShapeDtypeStruct((B_TOTAL, D_MODEL_K), jnp.float32),
    mesh=plsc.VectorSubcoreMesh(core_axis_name="c", subcore_axis_name="s"),
    compiler_params=pltpu.CompilerParams(use_tc_tiling_on_sc=False),
    scratch_types=[
        pltpu.VMEM((B_PER_W,), jnp.int32),
        pltpu.VMEM((CHUNK, D_MODEL_K), jnp.float32),
        pltpu.SemaphoreType.DMA,
    ],
)
def _emb_lookup(table_hbm, x_hbm, out_hbm, idx_v, rows_v, sem):
    wid = lax.axis_index("s") * NC + lax.axis_index("c")
    base = wid * B_PER_W
    pltpu.sync_copy(x_hbm.at[pl.ds(base, B_PER_W)], idx_v)

    for i in range(NITER):
        pltpu.async_copy(
            table_hbm.at[idx_v.at[pl.ds(i * CHUNK, CHUNK)]], rows_v, sem
        ).wait()

        def scale_row(r, carry):
            for d in range(D_MODEL_K // 16):
                sl = pl.ds(d * 16, 16)
                rows_v[r, sl] = rows_v[r, sl] * SCALE
            return carry

        lax.fori_loop(0, CHUNK, scale_row, 0)

        pltpu.sync_copy(rows_v, out_hbm.at[pl.ds(base + i * CHUNK, CHUNK)])


def kernel(x, table):
    xf = x.reshape(-1).astype(jnp.int32)
    out = _emb_lookup(table, xf)
    return out.reshape(x.shape + (D_MODEL_K,))


# speedup vs baseline: 1.1159x; 1.1159x over previous
import functools
import math

import jax
import jax.numpy as jnp
from jax import lax
from jax.experimental import pallas as pl
from jax.experimental.pallas import tpu as pltpu
from jax.experimental.pallas import tpu_sc as plsc

D_MODEL_K = 64
VOCAB_K = 1_000_000
SCALE = math.sqrt(D_MODEL_K)

NC = 2
NS = 16
NW = NC * NS

B_TOTAL = 4096 * 200
B_PER_W = B_TOTAL // NW
CHUNK = 512
NITER = B_PER_W // CHUNK


@functools.partial(
    pl.kernel,
    out_type=jax.ShapeDtypeStruct((B_TOTAL, D_MODEL_K), jnp.float32),
    mesh=plsc.VectorSubcoreMesh(core_axis_name="c", subcore_axis_name="s"),
    compiler_params=pltpu.CompilerParams(use_tc_tiling_on_sc=False),
    scratch_types=[
        pltpu.VMEM((B_PER_W,), jnp.int32),
        pltpu.VMEM((2, CHUNK, D_MODEL_K), jnp.float32),
        pltpu.SemaphoreType.DMA((2,)),
        pltpu.SemaphoreType.DMA((2,)),
    ],
)
def _emb_lookup(table_hbm, x_hbm, out_hbm, idx_v, rows_v, gsem, osem):
    wid = lax.axis_index("s") * NC + lax.axis_index("c")
    base = wid * B_PER_W
    pltpu.sync_copy(x_hbm.at[pl.ds(base, B_PER_W)], idx_v)

    def issue_gather(i, b):
        pltpu.async_copy(
            table_hbm.at[idx_v.at[pl.ds(i * CHUNK, CHUNK)]],
            rows_v.at[b],
            gsem.at[b],
        )

    def wait_gather(b):
        pltpu.make_async_copy(
            table_hbm.at[pl.ds(0, CHUNK)], rows_v.at[b], gsem.at[b]
        ).wait()

    def issue_scatter(i, b):
        pltpu.async_copy(
            rows_v.at[b], out_hbm.at[pl.ds(base + i * CHUNK, CHUNK)], osem.at[b]
        )

    def wait_scatter(b):
        pltpu.make_async_copy(
            rows_v.at[b], out_hbm.at[pl.ds(0, CHUNK)], osem.at[b]
        ).wait()

    def scale_buf(b):
        @plsc.parallel_loop(0, CHUNK, step=2, unroll=4)
        def _(r):
            for rr in range(2):
                for d in range(D_MODEL_K // 16):
                    sl = pl.ds(d * 16, 16)
                    rows_v[b, r + rr, sl] = rows_v[b, r + rr, sl] * SCALE

    issue_gather(0, 0)
    for i in range(1, NITER + 1):
        b, pb = i % 2, (i - 1) % 2
        if i < NITER:
            if i >= 2:
                wait_scatter(b)
            issue_gather(i, b)
        wait_gather(pb)
        scale_buf(pb)
        issue_scatter(i - 1, pb)
    wait_scatter(0)
    wait_scatter(1)


def kernel(x, table):
    xf = x.reshape(-1).astype(jnp.int32)
    out = _emb_lookup(table, xf)
    return out.reshape(x.shape + (D_MODEL_K,))
